# baseline (speedup 1.0000x reference)
import functools

import jax
import jax.numpy as jnp
from jax.experimental import pallas as pl
from jax.experimental.pallas import tpu as pltpu

N = 5000
K = 12544
H = 1024
BK = 256
TAIL_CHUNK = 200
OUT_W = 128


def _boxhead_kernel(x_ref, w1_ref, b1_ref, w2_ref, b2_ref, wh_ref, bh_ref,
                    out_ref, acc_ref):
    k = pl.program_id(0)
    nk = pl.num_programs(0)
    part = jnp.dot(x_ref[...].astype(jnp.bfloat16),
                   w1_ref[...].astype(jnp.bfloat16),
                   preferred_element_type=jnp.float32)

    @pl.when(k == 0)
    def _init():
        acc_ref[...] = part

    @pl.when(k > 0)
    def _acc():
        acc_ref[...] += part

    @pl.when(k == nk - 1)
    def _tail():
        w2 = w2_ref[...].astype(jnp.bfloat16)
        wh = wh_ref[...].astype(jnp.bfloat16)

        def body(i, _):
            rows = pl.ds(i * TAIL_CHUNK, TAIL_CHUNK)
            h1 = jnp.maximum(acc_ref[rows, :] + b1_ref[...], 0.0)
            h2 = jnp.dot(h1.astype(jnp.bfloat16), w2,
                         preferred_element_type=jnp.float32)
            h2 = jnp.maximum(h2 + b2_ref[...], 0.0)
            out = jnp.dot(h2.astype(jnp.bfloat16), wh,
                          preferred_element_type=jnp.float32)
            out_ref[rows, :] = out + bh_ref[...]
            return 0

        jax.lax.fori_loop(0, N // TAIL_CHUNK, body, 0)


@functools.partial(jax.jit, static_argnums=())
def kernel(feature_vectors, W1, b1, W2, b2, Wc, bc, Wr, br):
    n_heads = Wc.shape[1] + Wr.shape[1]
    wh = jnp.concatenate(
        [Wc, Wr, jnp.zeros((H, OUT_W - n_heads), dtype=Wc.dtype)], axis=1)
    bh = jnp.concatenate(
        [bc, br, jnp.zeros((OUT_W - n_heads,), dtype=bc.dtype)])

    grid = (K // BK,)
    out = pl.pallas_call(
        _boxhead_kernel,
        grid=grid,
        in_specs=[
            pl.BlockSpec((N, BK), lambda k: (0, k)),
            pl.BlockSpec((BK, H), lambda k: (k, 0)),
            pl.BlockSpec((1, H), lambda k: (0, 0)),
            pl.BlockSpec((H, H), lambda k: (0, 0)),
            pl.BlockSpec((1, H), lambda k: (0, 0)),
            pl.BlockSpec((H, OUT_W), lambda k: (0, 0)),
            pl.BlockSpec((1, OUT_W), lambda k: (0, 0)),
        ],
        out_specs=pl.BlockSpec((N, OUT_W), lambda k: (0, 0)),
        out_shape=jax.ShapeDtypeStruct((N, OUT_W), jnp.float32),
        scratch_shapes=[pltpu.VMEM((N, H), jnp.float32)],
        compiler_params=pltpu.CompilerParams(
            dimension_semantics=("arbitrary",),
        ),
    )(feature_vectors, W1, b1.reshape(1, H), W2, b2.reshape(1, H),
      wh, bh.reshape(1, OUT_W))

    return out[:, :Wc.shape[1]], out[:, Wc.shape[1]:n_heads]


# speedup vs baseline: 1.3299x; 1.3299x over previous
import jax
import jax.numpy as jnp
from jax.experimental import pallas as pl
from jax.experimental.pallas import tpu as pltpu

N = 5000
K = 12544
H = 1024
BM = 200
OUT_W = 128


def _boxhead_kernel(x_ref, w1_ref, b1_ref, w2_ref, b2_ref, wh_ref, bh_ref,
                    out_ref):
    x = x_ref[...].astype(jnp.bfloat16)
    h1 = jnp.dot(x, w1_ref[...], preferred_element_type=jnp.float32)
    h1 = jnp.maximum(h1 + b1_ref[...], 0.0)
    h2 = jnp.dot(h1.astype(jnp.bfloat16), w2_ref[...],
                 preferred_element_type=jnp.float32)
    h2 = jnp.maximum(h2 + b2_ref[...], 0.0)
    out = jnp.dot(h2.astype(jnp.bfloat16), wh_ref[...],
                  preferred_element_type=jnp.float32)
    out_ref[...] = out + bh_ref[...]


def kernel(feature_vectors, W1, b1, W2, b2, Wc, bc, Wr, br):
    n_heads = Wc.shape[1] + Wr.shape[1]
    wh = jnp.concatenate(
        [Wc, Wr, jnp.zeros((H, OUT_W - n_heads), dtype=Wc.dtype)], axis=1)
    bh = jnp.concatenate(
        [bc, br, jnp.zeros((OUT_W - n_heads,), dtype=bc.dtype)])

    w1b = W1.astype(jnp.bfloat16)
    w2b = W2.astype(jnp.bfloat16)
    whb = wh.astype(jnp.bfloat16)

    grid = (N // BM,)
    out = pl.pallas_call(
        _boxhead_kernel,
        grid=grid,
        in_specs=[
            pl.BlockSpec((BM, K), lambda m: (m, 0)),
            pl.BlockSpec((K, H), lambda m: (0, 0)),
            pl.BlockSpec((1, H), lambda m: (0, 0)),
            pl.BlockSpec((H, H), lambda m: (0, 0)),
            pl.BlockSpec((1, H), lambda m: (0, 0)),
            pl.BlockSpec((H, OUT_W), lambda m: (0, 0)),
            pl.BlockSpec((1, OUT_W), lambda m: (0, 0)),
        ],
        out_specs=pl.BlockSpec((BM, OUT_W), lambda m: (m, 0)),
        out_shape=jax.ShapeDtypeStruct((N, OUT_W), jnp.float32),
        compiler_params=pltpu.CompilerParams(
            dimension_semantics=("arbitrary",),
        ),
    )(feature_vectors, w1b, b1.reshape(1, H), w2b, b2.reshape(1, H),
      whb, bh.reshape(1, OUT_W))

    return out[:, :Wc.shape[1]], out[:, Wc.shape[1]:n_heads]
